# baseline (speedup 1.0000x reference)
import functools

import jax
import jax.numpy as jnp
from jax import lax
from jax.experimental import pallas as pl
from jax.experimental.pallas import tpu as pltpu
from jax.experimental.pallas import tpu_sc as plsc

NC = 2
NS = 16
NW = NC * NS

B = 4096
S = 50
D = 128
L = 16
CHUNK_ROWS = 2
CHUNK_IDX = CHUNK_ROWS * S
B_PER_W = B // NW
N_CHUNKS = B_PER_W // CHUNK_ROWS


def _sc_gather_means(x_r, emb):
    mesh = plsc.VectorSubcoreMesh(core_axis_name="c", subcore_axis_name="s")

    @functools.partial(
        pl.kernel,
        mesh=mesh,
        out_type=jax.ShapeDtypeStruct((NW, B_PER_W, D), jnp.float32),
        scratch_types=[
            pltpu.VMEM((N_CHUNKS, CHUNK_IDX), jnp.int32),
            pltpu.VMEM((CHUNK_IDX, D), jnp.float32),
            pltpu.VMEM((CHUNK_IDX, D), jnp.float32),
            pltpu.VMEM((CHUNK_IDX, D), jnp.float32),
            pltpu.VMEM((CHUNK_IDX, D), jnp.float32),
            pltpu.VMEM((CHUNK_IDX, D), jnp.float32),
            pltpu.VMEM((CHUNK_IDX, D), jnp.float32),
            pltpu.VMEM((B_PER_W, D), jnp.float32),
            pltpu.SemaphoreType.DMA,
            pltpu.SemaphoreType.DMA,
            pltpu.SemaphoreType.DMA,
            pltpu.SemaphoreType.DMA,
            pltpu.SemaphoreType.DMA,
            pltpu.SemaphoreType.DMA,
        ],
    )
    def means_kernel(x_hbm, emb_hbm, out_hbm, idx_v, buf0, buf1, buf2, buf3,
                     buf4, buf5, acc, sem0, sem1, sem2, sem3, sem4, sem5):
        wid = lax.axis_index("s") * NC + lax.axis_index("c")
        pltpu.sync_copy(x_hbm.at[wid], idx_v)

        def start(chunk, buf, sem):
            pltpu.make_async_copy(emb_hbm.at[idx_v.at[chunk]], buf, sem).start()

        def wait(chunk, buf, sem):
            pltpu.make_async_copy(emb_hbm.at[idx_v.at[chunk]], buf, sem).wait()

        def process(buf, j, row):
            for h in range(CHUNK_ROWS):
                base = h * S
                init = tuple(buf[base, pl.ds(c * L, L)] for c in range(D // L))

                def rbody(r, accs, base=base):
                    return tuple(accs[c] + buf[base + 1 + r, pl.ds(c * L, L)]
                                 for c in range(D // L))

                accs = lax.fori_loop(0, S - 1, rbody, init, unroll=7)
                for c in range(D // L):
                    acc[row + h, pl.ds(c * L, L)] = accs[c]

        NBUF = 6
        bufs = (buf0, buf1, buf2, buf3, buf4, buf5)
        sems = (sem0, sem1, sem2, sem3, sem4, sem5)
        for t in range(NBUF - 1):
            start(t, bufs[t], sems[t])

        def ring_body(j, carry):
            for t in range(NBUF):
                chunk = NBUF * j + t

                @pl.when(chunk + NBUF - 1 < N_CHUNKS)
                def _(t=t, chunk=chunk):
                    start(chunk + NBUF - 1, bufs[(t + NBUF - 1) % NBUF],
                          sems[(t + NBUF - 1) % NBUF])

                wait(chunk, bufs[t], sems[t])
                process(bufs[t], chunk, CHUNK_ROWS * chunk)
            return carry

        lax.fori_loop(0, N_CHUNKS // NBUF, ring_body, 0)
        for t in range(N_CHUNKS - N_CHUNKS % NBUF, N_CHUNKS):
            wait(t, bufs[t % NBUF], sems[t % NBUF])
            process(bufs[t % NBUF], t, CHUNK_ROWS * t)
        pltpu.sync_copy(acc, out_hbm.at[wid])

    return means_kernel(x_r, emb)


def _tc_finish(sums, x, W, b):
    BM = 1024

    def body(s_ref, x_ref, w_ref, b_ref, o_ref):
        cnt = jnp.sum((x_ref[...] != 0).astype(jnp.float32), axis=1, keepdims=True)
        denom = jnp.maximum(cnt, 1.0)
        z = s_ref[...] / denom
        y = lax.dot_general(z, w_ref[...], (((1,), (1,)), ((), ())),
                            preferred_element_type=jnp.float32)
        o_ref[...] = jnp.maximum(y + b_ref[...], 0.0)

    return pl.pallas_call(
        body,
        grid=(B // BM,),
        in_specs=[
            pl.BlockSpec((BM, D), lambda i: (i, 0)),
            pl.BlockSpec((BM, S), lambda i: (i, 0)),
            pl.BlockSpec((D, D), lambda i: (0, 0)),
            pl.BlockSpec((1, D), lambda i: (0, 0)),
        ],
        out_specs=pl.BlockSpec((BM, D), lambda i: (i, 0)),
        out_shape=jax.ShapeDtypeStruct((B, D), jnp.float32),
    )(sums, x, W, b.reshape(1, D))


def kernel(x, emb, W, b):
    x_r = x.reshape(NW, N_CHUNKS, CHUNK_IDX)
    sums = _sc_gather_means(x_r, emb).reshape(B, D)
    return _tc_finish(sums, x, W, b)


# speedup vs baseline: 1.0133x; 1.0133x over previous
import functools

import jax
import jax.numpy as jnp
from jax import lax
from jax.experimental import pallas as pl
from jax.experimental.pallas import tpu as pltpu
from jax.experimental.pallas import tpu_sc as plsc

NC = 2
NS = 16
NW = NC * NS

B = 4096
S = 50
D = 128
L = 16
CHUNK_ROWS = 2
CHUNK_IDX = CHUNK_ROWS * S
B_PER_W = B // NW
N_CHUNKS = B_PER_W // CHUNK_ROWS


def _sc_gather_means(x_r, emb):
    mesh = plsc.VectorSubcoreMesh(core_axis_name="c", subcore_axis_name="s")

    @functools.partial(
        pl.kernel,
        mesh=mesh,
        out_type=jax.ShapeDtypeStruct((NW, B_PER_W, D), jnp.float32),
        scratch_types=[
            pltpu.VMEM((N_CHUNKS, CHUNK_IDX), jnp.int32),
            pltpu.VMEM((CHUNK_IDX, D), jnp.float32),
            pltpu.VMEM((CHUNK_IDX, D), jnp.float32),
            pltpu.VMEM((CHUNK_IDX, D), jnp.float32),
            pltpu.VMEM((CHUNK_IDX, D), jnp.float32),
            pltpu.VMEM((CHUNK_IDX, D), jnp.float32),
            pltpu.VMEM((CHUNK_IDX, D), jnp.float32),
            pltpu.VMEM((CHUNK_IDX, D), jnp.float32),
            pltpu.VMEM((CHUNK_IDX, D), jnp.float32),
            pltpu.VMEM((B_PER_W, D), jnp.float32),
            pltpu.SemaphoreType.DMA,
            pltpu.SemaphoreType.DMA,
            pltpu.SemaphoreType.DMA,
            pltpu.SemaphoreType.DMA,
            pltpu.SemaphoreType.DMA,
            pltpu.SemaphoreType.DMA,
            pltpu.SemaphoreType.DMA,
            pltpu.SemaphoreType.DMA,
        ],
    )
    def means_kernel(x_hbm, emb_hbm, out_hbm, idx_v, buf0, buf1, buf2, buf3,
                     buf4, buf5, buf6, buf7, acc,
                     sem0, sem1, sem2, sem3, sem4, sem5, sem6, sem7):
        wid = lax.axis_index("s") * NC + lax.axis_index("c")
        pltpu.sync_copy(x_hbm.at[wid], idx_v)

        def start(chunk, buf, sem):
            pltpu.make_async_copy(emb_hbm.at[idx_v.at[chunk]], buf, sem).start()

        def wait(chunk, buf, sem):
            pltpu.make_async_copy(emb_hbm.at[idx_v.at[chunk]], buf, sem).wait()

        def process(buf, j, row):
            for h in range(CHUNK_ROWS):
                base = h * S
                init = tuple(buf[base, pl.ds(c * L, L)] for c in range(D // L))

                def rbody(r, accs, base=base):
                    return tuple(accs[c] + buf[base + 1 + r, pl.ds(c * L, L)]
                                 for c in range(D // L))

                accs = lax.fori_loop(0, S - 1, rbody, init, unroll=7)
                for c in range(D // L):
                    acc[row + h, pl.ds(c * L, L)] = accs[c]

        NBUF = 8
        bufs = (buf0, buf1, buf2, buf3, buf4, buf5, buf6, buf7)
        sems = (sem0, sem1, sem2, sem3, sem4, sem5, sem6, sem7)
        for t in range(NBUF - 1):
            start(t, bufs[t], sems[t])

        def ring_body(j, carry):
            for t in range(NBUF):
                chunk = NBUF * j + t

                @pl.when(chunk + NBUF - 1 < N_CHUNKS)
                def _(t=t, chunk=chunk):
                    start(chunk + NBUF - 1, bufs[(t + NBUF - 1) % NBUF],
                          sems[(t + NBUF - 1) % NBUF])

                wait(chunk, bufs[t], sems[t])
                process(bufs[t], chunk, CHUNK_ROWS * chunk)
            return carry

        lax.fori_loop(0, N_CHUNKS // NBUF, ring_body, 0)
        for t in range(N_CHUNKS - N_CHUNKS % NBUF, N_CHUNKS):
            wait(t, bufs[t % NBUF], sems[t % NBUF])
            process(bufs[t % NBUF], t, CHUNK_ROWS * t)
        pltpu.sync_copy(acc, out_hbm.at[wid])

    return means_kernel(x_r, emb)


def _tc_finish(sums, x, W, b):
    BM = 1024

    def body(s_ref, x_ref, w_ref, b_ref, o_ref):
        cnt = jnp.sum((x_ref[...] != 0).astype(jnp.float32), axis=1, keepdims=True)
        denom = jnp.maximum(cnt, 1.0)
        z = s_ref[...] / denom
        y = lax.dot_general(z, w_ref[...], (((1,), (1,)), ((), ())),
                            preferred_element_type=jnp.float32)
        o_ref[...] = jnp.maximum(y + b_ref[...], 0.0)

    return pl.pallas_call(
        body,
        grid=(B // BM,),
        in_specs=[
            pl.BlockSpec((BM, D), lambda i: (i, 0)),
            pl.BlockSpec((BM, S), lambda i: (i, 0)),
            pl.BlockSpec((D, D), lambda i: (0, 0)),
            pl.BlockSpec((1, D), lambda i: (0, 0)),
        ],
        out_specs=pl.BlockSpec((BM, D), lambda i: (i, 0)),
        out_shape=jax.ShapeDtypeStruct((B, D), jnp.float32),
    )(sums, x, W, b.reshape(1, D))


def kernel(x, emb, W, b):
    x_r = x.reshape(NW, N_CHUNKS, CHUNK_IDX)
    sums = _sc_gather_means(x_r, emb).reshape(B, D)
    return _tc_finish(sums, x, W, b)


# speedup vs baseline: 1.0692x; 1.0552x over previous
import functools

import jax
import jax.numpy as jnp
from jax import lax
from jax.experimental import pallas as pl
from jax.experimental.pallas import tpu as pltpu
from jax.experimental.pallas import tpu_sc as plsc

NC = 2
NS = 16
NW = NC * NS

B = 4096
S = 50
D = 128
L = 16
CHUNK_ROWS = 2
CHUNK_IDX = CHUNK_ROWS * S
B_PER_W = B // NW
N_CHUNKS = B_PER_W // CHUNK_ROWS


def _sc_gather_means(x_r, emb):
    mesh = plsc.VectorSubcoreMesh(core_axis_name="c", subcore_axis_name="s")

    @functools.partial(
        pl.kernel,
        mesh=mesh,
        out_type=jax.ShapeDtypeStruct((NW, B_PER_W, D), jnp.float32),
        scratch_types=[
            pltpu.VMEM((N_CHUNKS, CHUNK_IDX), jnp.int32),
            pltpu.VMEM((CHUNK_IDX, D), jnp.float32),
            pltpu.VMEM((CHUNK_IDX, D), jnp.float32),
            pltpu.VMEM((CHUNK_IDX, D), jnp.float32),
            pltpu.VMEM((CHUNK_IDX, D), jnp.float32),
            pltpu.VMEM((B_PER_W, D), jnp.float32),
            pltpu.SemaphoreType.DMA,
            pltpu.SemaphoreType.DMA,
            pltpu.SemaphoreType.DMA,
            pltpu.SemaphoreType.DMA,
        ],
    )
    def means_kernel(x_hbm, emb_hbm, out_hbm, idx_v, buf0, buf1, buf2, buf3,
                     acc, sem0, sem1, sem2, sem3):
        wid = lax.axis_index("s") * NC + lax.axis_index("c")
        pltpu.sync_copy(x_hbm.at[wid], idx_v)

        def start(chunk, buf, sem):
            pltpu.make_async_copy(emb_hbm.at[idx_v.at[chunk]], buf, sem).start()

        def wait(chunk, buf, sem):
            pltpu.make_async_copy(emb_hbm.at[idx_v.at[chunk]], buf, sem).wait()

        def process(buf, j, row):
            for h in range(CHUNK_ROWS):
                base = h * S
                init = tuple(buf[base, pl.ds(c * L, L)] for c in range(D // L))

                def rbody(r, accs, base=base):
                    return tuple(accs[c] + buf[base + 1 + r, pl.ds(c * L, L)]
                                 for c in range(D // L))

                accs = lax.fori_loop(0, S - 1, rbody, init, unroll=7)
                for c in range(D // L):
                    acc[row + h, pl.ds(c * L, L)] = accs[c]

        NBUF = 4
        bufs = (buf0, buf1, buf2, buf3)
        sems = (sem0, sem1, sem2, sem3)
        for t in range(NBUF - 1):
            start(t, bufs[t], sems[t])

        def ring_body(j, carry):
            for t in range(NBUF):
                chunk = NBUF * j + t

                @pl.when(chunk + NBUF - 1 < N_CHUNKS)
                def _(t=t, chunk=chunk):
                    start(chunk + NBUF - 1, bufs[(t + NBUF - 1) % NBUF],
                          sems[(t + NBUF - 1) % NBUF])

                wait(chunk, bufs[t], sems[t])
                process(bufs[t], chunk, CHUNK_ROWS * chunk)
            return carry

        lax.fori_loop(0, N_CHUNKS // NBUF, ring_body, 0)
        pltpu.sync_copy(acc, out_hbm.at[wid])

    return means_kernel(x_r, emb)


def _tc_finish(sums, x, W, b):
    BM = 1024

    def body(s_ref, x_ref, w_ref, b_ref, o_ref):
        cnt = jnp.sum((x_ref[...] != 0).astype(jnp.float32), axis=1, keepdims=True)
        denom = jnp.maximum(cnt, 1.0)
        z = s_ref[...] / denom
        y = lax.dot_general(z, w_ref[...], (((1,), (1,)), ((), ())),
                            preferred_element_type=jnp.float32)
        o_ref[...] = jnp.maximum(y + b_ref[...], 0.0)

    return pl.pallas_call(
        body,
        grid=(B // BM,),
        in_specs=[
            pl.BlockSpec((BM, D), lambda i: (i, 0)),
            pl.BlockSpec((BM, S), lambda i: (i, 0)),
            pl.BlockSpec((D, D), lambda i: (0, 0)),
            pl.BlockSpec((1, D), lambda i: (0, 0)),
        ],
        out_specs=pl.BlockSpec((BM, D), lambda i: (i, 0)),
        out_shape=jax.ShapeDtypeStruct((B, D), jnp.float32),
    )(sums, x, W, b.reshape(1, D))


def kernel(x, emb, W, b):
    x_r = x.reshape(NW, N_CHUNKS, CHUNK_IDX)
    sums = _sc_gather_means(x_r, emb).reshape(B, D)
    return _tc_finish(sums, x, W, b)
